# baseline (speedup 1.0000x reference)
import jax
import jax.numpy as jnp
import numpy as np
from jax import lax
from jax.experimental import pallas as pl
from jax.experimental.pallas import tpu as pltpu

R = 2
CB = 4


def _perm_matrix(L):
    P = np.zeros((L, L), dtype=np.float32)
    for l in range(L):
        p = (l & 64) | ((l & 1) << 5) | ((l & 63) >> 1)
        P[l, p] = 1.0
    return P


def _body(x_ref, p_ref, o_ref):
    for c4 in range(CB):
        for i in range(R):
            v = x_ref[0, c4, :, i]
            HH, WW, L = v.shape
            v2 = v.reshape(HH * WW, L)
            for q in range(4):
                r = jnp.dot(v2, p_ref[:, 32 * q:32 * (q + 1)],
                            preferred_element_type=jnp.float32,
                            precision=lax.Precision.HIGHEST)
                o_ref[0, c4, i, q] = r.reshape(HH, WW, 32)


def kernel(x):
    B, C, H, W, Z = x.shape
    L = R * Z
    G = (B * C) // CB
    xv = x.reshape(G, CB, H // R, R, W // R, L)
    P = jnp.asarray(_perm_matrix(L))
    out = pl.pallas_call(
        _body,
        grid=(G,),
        in_specs=[
            pl.BlockSpec((1, CB, H // R, R, W // R, L),
                         lambda g: (g, 0, 0, 0, 0, 0)),
            pl.BlockSpec((L, L), lambda g: (0, 0)),
        ],
        out_specs=pl.BlockSpec((1, CB, R, R * R, H // R, W // R, Z // R),
                               lambda g: (g, 0, 0, 0, 0, 0, 0)),
        out_shape=jax.ShapeDtypeStruct(
            (G, CB, R, R * R, H // R, W // R, Z // R), x.dtype),
    )(xv, P)
    return out.reshape(B, C * R**3, H // R, W // R, Z // R)


# speedup vs baseline: 1.3878x; 1.3878x over previous
import jax
import jax.numpy as jnp
import numpy as np
from jax import lax
from jax.experimental import pallas as pl
from jax.experimental.pallas import tpu as pltpu

R = 2
CB = 4


def _perm_matrix(L):
    P = np.zeros((L, L), dtype=np.float32)
    for l in range(L):
        p = (l & 64) | ((l & 1) << 5) | ((l & 63) >> 1)
        P[l, p] = 1.0
    return P


def _body(x_ref, p_ref, o_ref):
    pm = p_ref[...]
    for c4 in range(CB):
        for i in range(R):
            v = x_ref[0, c4, :, i]
            HH, WW, L = v.shape
            r = jnp.dot(v.reshape(HH * WW, L), pm,
                        preferred_element_type=jnp.float32,
                        precision=lax.Precision.HIGHEST)
            r = r.reshape(HH, WW, L)
            for q in range(4):
                o_ref[0, c4, i, q] = r[:, :, 32 * q:32 * (q + 1)]


def kernel(x):
    B, C, H, W, Z = x.shape
    L = R * Z
    G = (B * C) // CB
    xv = x.reshape(G, CB, H // R, R, W // R, L)
    P = jnp.asarray(_perm_matrix(L))
    out = pl.pallas_call(
        _body,
        grid=(G,),
        in_specs=[
            pl.BlockSpec((1, CB, H // R, R, W // R, L),
                         lambda g: (g, 0, 0, 0, 0, 0)),
            pl.BlockSpec((L, L), lambda g: (0, 0)),
        ],
        out_specs=pl.BlockSpec((1, CB, R, R * R, H // R, W // R, Z // R),
                               lambda g: (g, 0, 0, 0, 0, 0, 0)),
        out_shape=jax.ShapeDtypeStruct(
            (G, CB, R, R * R, H // R, W // R, Z // R), x.dtype),
    )(xv, P)
    return out.reshape(B, C * R**3, H // R, W // R, Z // R)
